# baseline (speedup 1.0000x reference)
import functools

import jax
import jax.numpy as jnp
from jax import lax
from jax.experimental import pallas as pl
from jax.experimental.pallas import tpu as pltpu
from jax.experimental.pallas import tpu_sc as plsc

N_NODES = 10000
D = 128
EPS = 1e-5

NC = 2
NS = 16
NW = NC * NS
CHUNK = 128
N_PAD = 10112
ROWS_PER_TILE = N_PAD // NS


def _make_sc_agg(e_pad: int):
    cpt = e_pad // (NW * CHUNK)
    H = cpt // 2
    assert cpt % 16 == 0
    mesh = plsc.VectorSubcoreMesh(
        core_axis_name="c", subcore_axis_name="s", num_cores=NC, num_subcores=NS
    )

    @functools.partial(
        pl.kernel,
        out_type=jax.ShapeDtypeStruct((NC, N_PAD, D), jnp.float32),
        mesh=mesh,
        scratch_types=[
            pltpu.VMEM((H, CHUNK), jnp.int32),
            pltpu.VMEM((H, CHUNK), jnp.int32),
            pltpu.VMEM((CHUNK, D), jnp.float32),
            pltpu.VMEM((CHUNK, D), jnp.float32),
            pltpu.VMEM_SHARED((N_PAD, D), jnp.float32),
            pltpu.SemaphoreType.DMA,
            pltpu.SemaphoreType.DMA,
            pltpu.SemaphoreType.DMA,
            pltpu.SemaphoreType.DMA,
        ],
    )
    def sc_agg(x_hbm, src_hbm, dst_hbm, zeros_hbm, out_hbm,
               idx_s, idx_d, rows0, rows1, agg_sh, gsem0, gsem1, ssem0, ssem1):
        c = lax.axis_index("c")
        s = lax.axis_index("s")
        wid = c * NS + s
        r0 = s * ROWS_PER_TILE
        rows = (rows0, rows1)
        gsem = (gsem0, gsem1)
        ssem = (ssem0, ssem1)

        def g_start(j, b):
            pltpu.async_copy(x_hbm.at[idx_s.at[j]], rows[b], gsem[b])

        def g_wait(j, b):
            pltpu.make_async_copy(x_hbm.at[idx_s.at[j]], rows[b],
                                  gsem[b]).wait()

        def s_start(j, b):
            pltpu.async_copy(rows[b], agg_sh.at[idx_d.at[j]], ssem[b],
                             add=True)

        def s_wait(j, b):
            pltpu.make_async_copy(rows[b], agg_sh.at[idx_d.at[j]],
                                  ssem[b]).wait()

        pltpu.sync_copy(zeros_hbm.at[pl.ds(r0, ROWS_PER_TILE)],
                        agg_sh.at[pl.ds(r0, ROWS_PER_TILE)])
        plsc.subcore_barrier()

        for h in range(2):
            base = wid * cpt + h * H
            pltpu.sync_copy(src_hbm.at[pl.ds(base, H)], idx_s)
            pltpu.sync_copy(dst_hbm.at[pl.ds(base, H)], idx_d)
            for j in range(H):
                g_start(j, 0)
                g_wait(j, 0)
                s_start(j, 0)
                s_wait(j, 0)

        plsc.subcore_barrier()
        pltpu.sync_copy(agg_sh.at[pl.ds(r0, ROWS_PER_TILE)],
                        out_hbm.at[c, pl.ds(r0, ROWS_PER_TILE)])

    return sc_agg


def _make_tc_layer(relu: bool, pad_out: bool):
    def body(p_ref, x_ref, wrel_ref, brel_ref, wroot_ref, gamma_ref, beta_ref,
             o_ref):
        agg = p_ref[0, :N_NODES, :] + p_ref[1, :N_NODES, :]
        x = x_ref[:N_NODES, :]
        h = jnp.dot(agg, wrel_ref[...], preferred_element_type=jnp.float32)
        h = h + jnp.dot(x, wroot_ref[...], preferred_element_type=jnp.float32)
        h = h + brel_ref[...]
        mu = jnp.mean(h, axis=0, keepdims=True)
        var = jnp.mean(jnp.square(h - mu), axis=0, keepdims=True)
        hn = (h - mu) * lax.rsqrt(var + EPS) * gamma_ref[...] + beta_ref[...]
        if relu:
            hn = jnp.maximum(hn, 0.0)
        if pad_out:
            o_ref[:N_NODES, :] = hn
            o_ref[N_NODES:, :] = jnp.zeros((N_PAD - N_NODES, D), jnp.float32)
        else:
            o_ref[...] = hn

    out_rows = N_PAD if pad_out else N_NODES
    return pl.pallas_call(
        body,
        out_shape=jax.ShapeDtypeStruct((out_rows, D), jnp.float32),
    )


def kernel(x, edge_index, W_rel1, b_rel1, W_root1, gamma1, beta1,
           W_rel2, b_rel2, W_root2, gamma2, beta2):
    src = edge_index[0].astype(jnp.int32)
    dst = edge_index[1].astype(jnp.int32)
    e = src.shape[0]
    grain = NW * CHUNK * 16
    e_pad = ((e + grain - 1) // grain) * grain
    src_p = jnp.full((e_pad,), N_NODES, jnp.int32).at[:e].set(src)
    dst_p = jnp.full((e_pad,), N_NODES, jnp.int32).at[:e].set(dst)
    src_p = src_p.reshape(e_pad // CHUNK, CHUNK)
    dst_p = dst_p.reshape(e_pad // CHUNK, CHUNK)

    x_pad = jnp.zeros((N_PAD, D), jnp.float32).at[:N_NODES].set(x)
    zeros = jnp.zeros((N_PAD, D), jnp.float32)

    sc_agg = _make_sc_agg(e_pad)
    tc1 = _make_tc_layer(relu=True, pad_out=True)
    tc2 = _make_tc_layer(relu=False, pad_out=False)

    b1 = b_rel1.reshape(1, D)
    g1 = gamma1.reshape(1, D)
    be1 = beta1.reshape(1, D)
    b2 = b_rel2.reshape(1, D)
    g2 = gamma2.reshape(1, D)
    be2 = beta2.reshape(1, D)

    p1 = sc_agg(x_pad, src_p, dst_p, zeros)
    h1 = tc1(p1, x_pad, W_rel1, b1, W_root1, g1, be1)
    p2 = sc_agg(h1, src_p, dst_p, zeros)
    h2 = tc2(p2, h1, W_rel2, b2, W_root2, g2, be2)
    return h2


# speedup vs baseline: 2.1362x; 2.1362x over previous
import functools

import jax
import jax.numpy as jnp
from jax import lax
from jax.experimental import pallas as pl
from jax.experimental.pallas import tpu as pltpu
from jax.experimental.pallas import tpu_sc as plsc

N_NODES = 10000
D = 128
EPS = 1e-5

NC = 2
NS = 16
NW = NC * NS
CHUNK = 112
NR = 3
NI = 6
N_PAD = 10112
ROWS_PER_TILE = N_PAD // NS


def _make_sc_agg(e_pad: int):
    cpt = e_pad // (NW * CHUNK)
    ept = cpt * CHUNK
    mesh = plsc.VectorSubcoreMesh(
        core_axis_name="c", subcore_axis_name="s", num_cores=NC, num_subcores=NS
    )

    scratch = (
        [pltpu.VMEM((CHUNK,), jnp.int32) for _ in range(NI)]
        + [pltpu.VMEM((CHUNK,), jnp.int32) for _ in range(NI)]
        + [pltpu.VMEM((CHUNK, D), jnp.float32) for _ in range(NR)]
        + [pltpu.VMEM_SHARED((N_PAD, D), jnp.float32)]
        + [pltpu.SemaphoreType.DMA] * (2 * NI + 2 * NR)
    )

    @functools.partial(
        pl.kernel,
        out_type=jax.ShapeDtypeStruct((NC, N_PAD, D), jnp.float32),
        mesh=mesh,
        scratch_types=scratch,
    )
    def sc_agg(x_hbm, src_hbm, dst_hbm, zeros_hbm, out_hbm, *refs):
        idx_s = refs[0:NI]
        idx_d = refs[NI:2 * NI]
        rows = refs[2 * NI:2 * NI + NR]
        agg_sh = refs[2 * NI + NR]
        sems = refs[2 * NI + NR + 1:]
        isem_s = sems[0:NI]
        isem_d = sems[NI:2 * NI]
        gsem = sems[2 * NI:2 * NI + NR]
        ssem = sems[2 * NI + NR:]

        c = lax.axis_index("c")
        s = lax.axis_index("s")
        wid = c * NS + s
        r0 = s * ROWS_PER_TILE
        ebase = wid * ept

        def i_start(j):
            q = j % NI
            off = ebase + j * CHUNK
            pltpu.async_copy(src_hbm.at[pl.ds(off, CHUNK)], idx_s[q],
                             isem_s[q])
            pltpu.async_copy(dst_hbm.at[pl.ds(off, CHUNK)], idx_d[q],
                             isem_d[q])

        def i_wait(j):
            q = j % NI
            off = ebase + j * CHUNK
            pltpu.make_async_copy(src_hbm.at[pl.ds(off, CHUNK)], idx_s[q],
                                  isem_s[q]).wait()
            pltpu.make_async_copy(dst_hbm.at[pl.ds(off, CHUNK)], idx_d[q],
                                  isem_d[q]).wait()

        def g_start(j):
            pltpu.async_copy(x_hbm.at[idx_s[j % NI]], rows[j % NR],
                             gsem[j % NR])

        def g_wait(j):
            pltpu.make_async_copy(x_hbm.at[idx_s[j % NI]], rows[j % NR],
                                  gsem[j % NR]).wait()

        def s_start(j):
            pltpu.async_copy(rows[j % NR], agg_sh.at[idx_d[j % NI]],
                             ssem[j % NR], add=True)

        def s_wait(j):
            pltpu.make_async_copy(rows[j % NR], agg_sh.at[idx_d[j % NI]],
                                  ssem[j % NR]).wait()

        for j in range(min(NR, cpt)):
            i_start(j)
        pltpu.sync_copy(zeros_hbm.at[pl.ds(r0, ROWS_PER_TILE)],
                        agg_sh.at[pl.ds(r0, ROWS_PER_TILE)])
        plsc.subcore_barrier()

        for j in range(cpt):
            if j >= NR:
                s_wait(j - NR)
            if j + NR < cpt:
                i_start(j + NR)
            i_wait(j)
            g_start(j)
            if j >= 1:
                g_wait(j - 1)
                s_start(j - 1)
        g_wait(cpt - 1)
        s_start(cpt - 1)
        for j in range(max(cpt - NR, 0), cpt):
            s_wait(j)

        plsc.subcore_barrier()
        pltpu.sync_copy(agg_sh.at[pl.ds(r0, ROWS_PER_TILE)],
                        out_hbm.at[c, pl.ds(r0, ROWS_PER_TILE)])

    return sc_agg


def _make_tc_layer(relu: bool, pad_out: bool):
    def body(p_ref, x_ref, wrel_ref, brel_ref, wroot_ref, gamma_ref, beta_ref,
             o_ref):
        agg = p_ref[0, :N_NODES, :] + p_ref[1, :N_NODES, :]
        x = x_ref[:N_NODES, :]
        h = jnp.dot(agg, wrel_ref[...], preferred_element_type=jnp.float32)
        h = h + jnp.dot(x, wroot_ref[...], preferred_element_type=jnp.float32)
        h = h + brel_ref[...]
        mu = jnp.mean(h, axis=0, keepdims=True)
        var = jnp.mean(jnp.square(h - mu), axis=0, keepdims=True)
        hn = (h - mu) * lax.rsqrt(var + EPS) * gamma_ref[...] + beta_ref[...]
        if relu:
            hn = jnp.maximum(hn, 0.0)
        if pad_out:
            o_ref[:N_NODES, :] = hn
            o_ref[N_NODES:, :] = jnp.zeros((N_PAD - N_NODES, D), jnp.float32)
        else:
            o_ref[...] = hn

    out_rows = N_PAD if pad_out else N_NODES
    return pl.pallas_call(
        body,
        out_shape=jax.ShapeDtypeStruct((out_rows, D), jnp.float32),
    )


def kernel(x, edge_index, W_rel1, b_rel1, W_root1, gamma1, beta1,
           W_rel2, b_rel2, W_root2, gamma2, beta2):
    src = edge_index[0].astype(jnp.int32)
    dst = edge_index[1].astype(jnp.int32)
    e = src.shape[0]
    grain = NW * CHUNK
    e_pad = ((e + grain - 1) // grain) * grain
    src_p = jnp.full((e_pad,), N_NODES, jnp.int32).at[:e].set(src)
    dst_p = jnp.full((e_pad,), N_NODES, jnp.int32).at[:e].set(dst)

    x_pad = jnp.zeros((N_PAD, D), jnp.float32).at[:N_NODES].set(x)
    zeros = jnp.zeros((N_PAD, D), jnp.float32)

    sc_agg = _make_sc_agg(e_pad)
    tc1 = _make_tc_layer(relu=True, pad_out=True)
    tc2 = _make_tc_layer(relu=False, pad_out=False)

    b1 = b_rel1.reshape(1, D)
    g1 = gamma1.reshape(1, D)
    be1 = beta1.reshape(1, D)
    b2 = b_rel2.reshape(1, D)
    g2 = gamma2.reshape(1, D)
    be2 = beta2.reshape(1, D)

    p1 = sc_agg(x_pad, src_p, dst_p, zeros)
    h1 = tc1(p1, x_pad, W_rel1, b1, W_root1, g1, be1)
    p2 = sc_agg(h1, src_p, dst_p, zeros)
    h2 = tc2(p2, h1, W_rel2, b2, W_root2, g2, be2)
    return h2


# speedup vs baseline: 3.9733x; 1.8600x over previous
import functools

import jax
import jax.numpy as jnp
from jax import lax
from jax.experimental import pallas as pl
from jax.experimental.pallas import tpu as pltpu
from jax.experimental.pallas import tpu_sc as plsc

N_NODES = 10000
D = 128
EPS = 1e-5

NC = 2
NS = 16
NW = NC * NS
CHUNK = 112
NR = 3
NI = 6
N_PAD = 10112
ROWS_PER_TILE = N_PAD // NS


def _make_sc_agg(e_pad: int):
    cpt = e_pad // (NW * CHUNK)
    ept = cpt * CHUNK
    mesh = plsc.VectorSubcoreMesh(
        core_axis_name="c", subcore_axis_name="s", num_cores=NC, num_subcores=NS
    )

    scratch = (
        [pltpu.VMEM((CHUNK,), jnp.int32) for _ in range(NI)]
        + [pltpu.VMEM((CHUNK,), jnp.int32) for _ in range(NI)]
        + [pltpu.VMEM((CHUNK, D), jnp.float32) for _ in range(NR)]
        + [pltpu.VMEM_SHARED((N_PAD, D), jnp.float32)]
        + [pltpu.SemaphoreType.DMA] * (2 * NI + 2 * NR)
    )

    @functools.partial(
        pl.kernel,
        out_type=jax.ShapeDtypeStruct((NC, N_PAD, D), jnp.float32),
        mesh=mesh,
        scratch_types=scratch,
    )
    def sc_agg(x_hbm, src_hbm, dst_hbm, zeros_hbm, out_hbm, *refs):
        idx_s = refs[0:NI]
        idx_d = refs[NI:2 * NI]
        rows = refs[2 * NI:2 * NI + NR]
        agg_sh = refs[2 * NI + NR]
        sems = refs[2 * NI + NR + 1:]
        isem_s = sems[0:NI]
        isem_d = sems[NI:2 * NI]
        gsem = sems[2 * NI:2 * NI + NR]
        ssem = sems[2 * NI + NR:]

        c = lax.axis_index("c")
        s = lax.axis_index("s")
        wid = c * NS + s
        r0 = s * ROWS_PER_TILE
        ebase = wid * ept

        def i_start(j):
            q = j % NI
            off = ebase + j * CHUNK
            pltpu.async_copy(src_hbm.at[pl.ds(off, CHUNK)], idx_s[q],
                             isem_s[q])
            pltpu.async_copy(dst_hbm.at[pl.ds(off, CHUNK)], idx_d[q],
                             isem_d[q])

        def i_wait(j):
            q = j % NI
            off = ebase + j * CHUNK
            pltpu.make_async_copy(src_hbm.at[pl.ds(off, CHUNK)], idx_s[q],
                                  isem_s[q]).wait()
            pltpu.make_async_copy(dst_hbm.at[pl.ds(off, CHUNK)], idx_d[q],
                                  isem_d[q]).wait()

        def g_start(j):
            pltpu.async_copy(x_hbm.at[idx_s[j % NI]], rows[j % NR],
                             gsem[j % NR])

        def g_wait(j):
            pltpu.make_async_copy(x_hbm.at[idx_s[j % NI]], rows[j % NR],
                                  gsem[j % NR]).wait()

        def s_start(j):
            pltpu.async_copy(rows[j % NR], agg_sh.at[idx_d[j % NI]],
                             ssem[j % NR], add=True)

        def s_wait(j):
            pltpu.make_async_copy(rows[j % NR], agg_sh.at[idx_d[j % NI]],
                                  ssem[j % NR]).wait()

        for j in range(min(NR, cpt)):
            i_start(j)
        pltpu.sync_copy(zeros_hbm.at[pl.ds(r0, ROWS_PER_TILE)],
                        agg_sh.at[pl.ds(r0, ROWS_PER_TILE)])
        plsc.subcore_barrier()

        for j in range(cpt):
            if j >= NR:
                s_wait(j - NR)
            if j + NR < cpt:
                i_start(j + NR)
            i_wait(j)
            g_start(j)
            if j >= 1:
                g_wait(j - 1)
                s_start(j - 1)
        g_wait(cpt - 1)
        s_start(cpt - 1)
        for j in range(max(cpt - NR, 0), cpt):
            s_wait(j)

        plsc.subcore_barrier()
        pltpu.sync_copy(agg_sh.at[pl.ds(r0, ROWS_PER_TILE)],
                        out_hbm.at[c, pl.ds(r0, ROWS_PER_TILE)])

    return sc_agg


def _make_tc_layer(relu: bool, pad_out: bool):
    def body(p_ref, x_ref, wrel_ref, brel_ref, wroot_ref, gamma_ref, beta_ref,
             o_ref):
        agg = p_ref[0, :N_NODES, :] + p_ref[1, :N_NODES, :]
        x = x_ref[:N_NODES, :]
        h = jnp.dot(agg, wrel_ref[...], preferred_element_type=jnp.float32)
        h = h + jnp.dot(x, wroot_ref[...], preferred_element_type=jnp.float32)
        h = h + brel_ref[...]
        mu = jnp.mean(h, axis=0, keepdims=True)
        var = jnp.mean(jnp.square(h - mu), axis=0, keepdims=True)
        hn = (h - mu) * lax.rsqrt(var + EPS) * gamma_ref[...] + beta_ref[...]
        if relu:
            hn = jnp.maximum(hn, 0.0)
        if pad_out:
            o_ref[:N_NODES, :] = hn
            o_ref[N_NODES:, :] = jnp.zeros((N_PAD - N_NODES, D), jnp.float32)
        else:
            o_ref[...] = hn

    out_rows = N_PAD if pad_out else N_NODES
    return pl.pallas_call(
        body,
        out_shape=jax.ShapeDtypeStruct((out_rows, D), jnp.float32),
    )


def kernel(x, edge_index, W_rel1, b_rel1, W_root1, gamma1, beta1,
           W_rel2, b_rel2, W_root2, gamma2, beta2):
    src = edge_index[0].astype(jnp.int32)
    dst = edge_index[1].astype(jnp.int32)
    e = src.shape[0]
    grain = NW * CHUNK
    e_pad = ((e + grain - 1) // grain) * grain
    pad_ids = N_NODES + (jnp.arange(e_pad - e, dtype=jnp.int32)
                         % (N_PAD - N_NODES))
    src_p = jnp.concatenate([src, pad_ids])
    dst_p = jnp.concatenate([dst, pad_ids])

    x_pad = jnp.zeros((N_PAD, D), jnp.float32).at[:N_NODES].set(x)
    zeros = jnp.zeros((N_PAD, D), jnp.float32)

    sc_agg = _make_sc_agg(e_pad)
    tc1 = _make_tc_layer(relu=True, pad_out=True)
    tc2 = _make_tc_layer(relu=False, pad_out=False)

    b1 = b_rel1.reshape(1, D)
    g1 = gamma1.reshape(1, D)
    be1 = beta1.reshape(1, D)
    b2 = b_rel2.reshape(1, D)
    g2 = gamma2.reshape(1, D)
    be2 = beta2.reshape(1, D)

    p1 = sc_agg(x_pad, src_p, dst_p, zeros)
    h1 = tc1(p1, x_pad, W_rel1, b1, W_root1, g1, be1)
    p2 = sc_agg(h1, src_p, dst_p, zeros)
    h2 = tc2(p2, h1, W_rel2, b2, W_root2, g2, be2)
    return h2


# speedup vs baseline: 4.0142x; 1.0103x over previous
import functools

import jax
import jax.numpy as jnp
from jax import lax
from jax.experimental import pallas as pl
from jax.experimental.pallas import tpu as pltpu
from jax.experimental.pallas import tpu_sc as plsc

N_NODES = 10000
D = 128
EPS = 1e-5

NC = 2
NS = 16
NW = NC * NS
CHUNK = 96
NR = 4
NI = 6
SLAG = 2
N_PAD = 10048
ROW_SPLIT = 632
ROW_LAST = N_PAD - 15 * ROW_SPLIT


def _make_sc_agg(e_pad: int):
    cpt = e_pad // (NW * CHUNK)
    ept = cpt * CHUNK
    mesh = plsc.VectorSubcoreMesh(
        core_axis_name="c", subcore_axis_name="s", num_cores=NC, num_subcores=NS
    )

    scratch = (
        [pltpu.VMEM((CHUNK,), jnp.int32) for _ in range(NI)]
        + [pltpu.VMEM((CHUNK,), jnp.int32) for _ in range(NI)]
        + [pltpu.VMEM((CHUNK, D), jnp.float32) for _ in range(NR)]
        + [pltpu.VMEM_SHARED((N_PAD, D), jnp.float32)]
        + [pltpu.SemaphoreType.DMA] * (2 * NI + 2 * NR)
    )

    @functools.partial(
        pl.kernel,
        out_type=jax.ShapeDtypeStruct((NC, N_PAD, D), jnp.float32),
        mesh=mesh,
        scratch_types=scratch,
    )
    def sc_agg(x_hbm, src_hbm, dst_hbm, zeros_hbm, out_hbm, *refs):
        idx_s = refs[0:NI]
        idx_d = refs[NI:2 * NI]
        rows = refs[2 * NI:2 * NI + NR]
        agg_sh = refs[2 * NI + NR]
        sems = refs[2 * NI + NR + 1:]
        isem_s = sems[0:NI]
        isem_d = sems[NI:2 * NI]
        gsem = sems[2 * NI:2 * NI + NR]
        ssem = sems[2 * NI + NR:]

        c = lax.axis_index("c")
        s = lax.axis_index("s")
        wid = c * NS + s
        ebase = wid * ept

        def acc_copy(to_out: bool):
            def do(start, size):
                if to_out:
                    pltpu.sync_copy(agg_sh.at[pl.ds(start, size)],
                                    out_hbm.at[c, pl.ds(start, size)])
                else:
                    pltpu.sync_copy(zeros_hbm.at[pl.ds(start, size)],
                                    agg_sh.at[pl.ds(start, size)])

            @pl.when(s < NS - 1)
            def _():
                do(s * ROW_SPLIT, ROW_SPLIT)

            @pl.when(s == NS - 1)
            def _():
                do((NS - 1) * ROW_SPLIT, ROW_LAST)

        def i_start(j):
            q = j % NI
            off = ebase + j * CHUNK
            pltpu.async_copy(src_hbm.at[pl.ds(off, CHUNK)], idx_s[q],
                             isem_s[q])
            pltpu.async_copy(dst_hbm.at[pl.ds(off, CHUNK)], idx_d[q],
                             isem_d[q])

        def i_wait(j):
            q = j % NI
            off = ebase + j * CHUNK
            pltpu.make_async_copy(src_hbm.at[pl.ds(off, CHUNK)], idx_s[q],
                                  isem_s[q]).wait()
            pltpu.make_async_copy(dst_hbm.at[pl.ds(off, CHUNK)], idx_d[q],
                                  isem_d[q]).wait()

        def g_start(j):
            pltpu.async_copy(x_hbm.at[idx_s[j % NI]], rows[j % NR],
                             gsem[j % NR])

        def g_wait(j):
            pltpu.make_async_copy(x_hbm.at[idx_s[j % NI]], rows[j % NR],
                                  gsem[j % NR]).wait()

        def s_start(j):
            pltpu.async_copy(rows[j % NR], agg_sh.at[idx_d[j % NI]],
                             ssem[j % NR], add=True)

        def s_wait(j):
            pltpu.make_async_copy(rows[j % NR], agg_sh.at[idx_d[j % NI]],
                                  ssem[j % NR]).wait()

        for j in range(min(SLAG, cpt)):
            i_start(j)
        acc_copy(to_out=False)
        plsc.subcore_barrier()

        for j in range(cpt):
            if j >= NR:
                s_wait(j - NR)
            if j + SLAG < cpt:
                i_start(j + SLAG)
            i_wait(j)
            g_start(j)
            if j >= SLAG:
                g_wait(j - SLAG)
                s_start(j - SLAG)
        for j in range(max(cpt - SLAG, 0), cpt):
            g_wait(j)
            s_start(j)
        for j in range(max(cpt - NR, 0), cpt):
            s_wait(j)

        plsc.subcore_barrier()
        acc_copy(to_out=True)

    return sc_agg


def _make_tc_layer(relu: bool, pad_out: bool):
    def body(p_ref, x_ref, wrel_ref, brel_ref, wroot_ref, gamma_ref, beta_ref,
             o_ref):
        agg = p_ref[0, :N_NODES, :] + p_ref[1, :N_NODES, :]
        x = x_ref[:N_NODES, :]
        h = jnp.dot(agg, wrel_ref[...], preferred_element_type=jnp.float32)
        h = h + jnp.dot(x, wroot_ref[...], preferred_element_type=jnp.float32)
        h = h + brel_ref[...]
        mu = jnp.mean(h, axis=0, keepdims=True)
        var = jnp.mean(jnp.square(h - mu), axis=0, keepdims=True)
        hn = (h - mu) * lax.rsqrt(var + EPS) * gamma_ref[...] + beta_ref[...]
        if relu:
            hn = jnp.maximum(hn, 0.0)
        if pad_out:
            o_ref[:N_NODES, :] = hn
            o_ref[N_NODES:, :] = jnp.zeros((N_PAD - N_NODES, D), jnp.float32)
        else:
            o_ref[...] = hn

    out_rows = N_PAD if pad_out else N_NODES
    return pl.pallas_call(
        body,
        out_shape=jax.ShapeDtypeStruct((out_rows, D), jnp.float32),
    )


def kernel(x, edge_index, W_rel1, b_rel1, W_root1, gamma1, beta1,
           W_rel2, b_rel2, W_root2, gamma2, beta2):
    src = edge_index[0].astype(jnp.int32)
    dst = edge_index[1].astype(jnp.int32)
    e = src.shape[0]
    grain = NW * CHUNK
    e_pad = ((e + grain - 1) // grain) * grain
    pad_ids = N_NODES + (jnp.arange(e_pad - e, dtype=jnp.int32)
                         % (N_PAD - N_NODES))
    src_p = jnp.concatenate([src, pad_ids])
    dst_p = jnp.concatenate([dst, pad_ids])

    x_pad = jnp.zeros((N_PAD, D), jnp.float32).at[:N_NODES].set(x)
    zeros = jnp.zeros((N_PAD, D), jnp.float32)

    sc_agg = _make_sc_agg(e_pad)
    tc1 = _make_tc_layer(relu=True, pad_out=True)
    tc2 = _make_tc_layer(relu=False, pad_out=False)

    b1 = b_rel1.reshape(1, D)
    g1 = gamma1.reshape(1, D)
    be1 = beta1.reshape(1, D)
    b2 = b_rel2.reshape(1, D)
    g2 = gamma2.reshape(1, D)
    be2 = beta2.reshape(1, D)

    p1 = sc_agg(x_pad, src_p, dst_p, zeros)
    h1 = tc1(p1, x_pad, W_rel1, b1, W_root1, g1, be1)
    p2 = sc_agg(h1, src_p, dst_p, zeros)
    h2 = tc2(p2, h1, W_rel2, b2, W_root2, g2, be2)
    return h2


# speedup vs baseline: 4.0206x; 1.0016x over previous
import functools

import jax
import jax.numpy as jnp
from jax import lax
from jax.experimental import pallas as pl
from jax.experimental.pallas import tpu as pltpu
from jax.experimental.pallas import tpu_sc as plsc

N_NODES = 10000
D = 128
EPS = 1e-5

NC = 2
NS = 16
NW = NC * NS
CHUNK = 80
NR = 4
NI = 6
SLAG = 2
ROW_SPLIT = 632
ROW_LAST = N_NODES - (NS - 1) * ROW_SPLIT


def _make_sc_agg(e: int):
    assert e % (NW * CHUNK) == 0
    cpt = e // (NW * CHUNK)
    ept = cpt * CHUNK
    mesh = plsc.VectorSubcoreMesh(
        core_axis_name="c", subcore_axis_name="s", num_cores=NC, num_subcores=NS
    )

    scratch = (
        [pltpu.VMEM((CHUNK,), jnp.int32) for _ in range(NI)]
        + [pltpu.VMEM((CHUNK,), jnp.int32) for _ in range(NI)]
        + [pltpu.VMEM((CHUNK, D), jnp.float32) for _ in range(NR)]
        + [pltpu.VMEM_SHARED((N_NODES, D), jnp.float32)]
        + [pltpu.SemaphoreType.DMA] * (2 * NI + 2 * NR)
    )

    @functools.partial(
        pl.kernel,
        out_type=jax.ShapeDtypeStruct((NC, N_NODES, D), jnp.float32),
        mesh=mesh,
        scratch_types=scratch,
    )
    def sc_agg(x_hbm, src_hbm, dst_hbm, zeros_hbm, out_hbm, *refs):
        idx_s = refs[0:NI]
        idx_d = refs[NI:2 * NI]
        rows = refs[2 * NI:2 * NI + NR]
        agg_sh = refs[2 * NI + NR]
        sems = refs[2 * NI + NR + 1:]
        isem_s = sems[0:NI]
        isem_d = sems[NI:2 * NI]
        gsem = sems[2 * NI:2 * NI + NR]
        ssem = sems[2 * NI + NR:]

        c = lax.axis_index("c")
        s = lax.axis_index("s")
        wid = c * NS + s
        ebase = wid * ept

        def acc_zero():
            @pl.when(s < NS - 1)
            def _():
                pltpu.sync_copy(zeros_hbm.at[pl.ds(0, ROW_SPLIT)],
                                agg_sh.at[pl.ds(s * ROW_SPLIT, ROW_SPLIT)])

            @pl.when(s == NS - 1)
            def _():
                pltpu.sync_copy(zeros_hbm.at[pl.ds(0, ROW_LAST)],
                                agg_sh.at[pl.ds((NS - 1) * ROW_SPLIT,
                                                ROW_LAST)])

        def acc_out():
            @pl.when(s < NS - 1)
            def _():
                pltpu.sync_copy(agg_sh.at[pl.ds(s * ROW_SPLIT, ROW_SPLIT)],
                                out_hbm.at[c, pl.ds(s * ROW_SPLIT, ROW_SPLIT)])

            @pl.when(s == NS - 1)
            def _():
                pltpu.sync_copy(agg_sh.at[pl.ds((NS - 1) * ROW_SPLIT,
                                                ROW_LAST)],
                                out_hbm.at[c, pl.ds((NS - 1) * ROW_SPLIT,
                                                    ROW_LAST)])

        def i_start(j):
            q = j % NI
            off = ebase + j * CHUNK
            pltpu.async_copy(src_hbm.at[pl.ds(off, CHUNK)], idx_s[q],
                             isem_s[q])
            pltpu.async_copy(dst_hbm.at[pl.ds(off, CHUNK)], idx_d[q],
                             isem_d[q])

        def i_wait(j):
            q = j % NI
            off = ebase + j * CHUNK
            pltpu.make_async_copy(src_hbm.at[pl.ds(off, CHUNK)], idx_s[q],
                                  isem_s[q]).wait()
            pltpu.make_async_copy(dst_hbm.at[pl.ds(off, CHUNK)], idx_d[q],
                                  isem_d[q]).wait()

        def g_start(j):
            pltpu.async_copy(x_hbm.at[idx_s[j % NI]], rows[j % NR],
                             gsem[j % NR])

        def g_wait(j):
            pltpu.make_async_copy(x_hbm.at[idx_s[j % NI]], rows[j % NR],
                                  gsem[j % NR]).wait()

        def s_start(j):
            pltpu.async_copy(rows[j % NR], agg_sh.at[idx_d[j % NI]],
                             ssem[j % NR], add=True)

        def s_wait(j):
            pltpu.make_async_copy(rows[j % NR], agg_sh.at[idx_d[j % NI]],
                                  ssem[j % NR]).wait()

        for j in range(min(SLAG, cpt)):
            i_start(j)
        acc_zero()
        plsc.subcore_barrier()

        for j in range(cpt):
            if j >= NR:
                s_wait(j - NR)
            if j + SLAG < cpt:
                i_start(j + SLAG)
            i_wait(j)
            g_start(j)
            if j >= SLAG:
                g_wait(j - SLAG)
                s_start(j - SLAG)
        for j in range(max(cpt - SLAG, 0), cpt):
            g_wait(j)
            s_start(j)
        for j in range(max(cpt - NR, 0), cpt):
            s_wait(j)

        plsc.subcore_barrier()
        acc_out()

    return sc_agg


def _make_tc_layer(relu: bool):
    def body(p_ref, x_ref, wrel_ref, brel_ref, wroot_ref, gamma_ref, beta_ref,
             o_ref):
        agg = p_ref[0] + p_ref[1]
        h = jnp.dot(agg, wrel_ref[...], preferred_element_type=jnp.float32)
        h = h + jnp.dot(x_ref[...], wroot_ref[...],
                        preferred_element_type=jnp.float32)
        h = h + brel_ref[...]
        mu = jnp.mean(h, axis=0, keepdims=True)
        var = jnp.mean(jnp.square(h - mu), axis=0, keepdims=True)
        hn = (h - mu) * lax.rsqrt(var + EPS) * gamma_ref[...] + beta_ref[...]
        if relu:
            hn = jnp.maximum(hn, 0.0)
        o_ref[...] = hn

    return pl.pallas_call(
        body,
        out_shape=jax.ShapeDtypeStruct((N_NODES, D), jnp.float32),
    )


def kernel(x, edge_index, W_rel1, b_rel1, W_root1, gamma1, beta1,
           W_rel2, b_rel2, W_root2, gamma2, beta2):
    src = edge_index[0].astype(jnp.int32)
    dst = edge_index[1].astype(jnp.int32)
    e = src.shape[0]

    zeros = jnp.zeros((ROW_SPLIT, D), jnp.float32)

    sc_agg = _make_sc_agg(e)
    tc1 = _make_tc_layer(relu=True)
    tc2 = _make_tc_layer(relu=False)

    b1 = b_rel1.reshape(1, D)
    g1 = gamma1.reshape(1, D)
    be1 = beta1.reshape(1, D)
    b2 = b_rel2.reshape(1, D)
    g2 = gamma2.reshape(1, D)
    be2 = beta2.reshape(1, D)

    p1 = sc_agg(x, src, dst, zeros)
    h1 = tc1(p1, x, W_rel1, b1, W_root1, g1, be1)
    p2 = sc_agg(h1, src, dst, zeros)
    h2 = tc2(p2, h1, W_rel2, b2, W_root2, g2, be2)
    return h2


# speedup vs baseline: 4.1416x; 1.0301x over previous
import functools

import jax
import jax.numpy as jnp
from jax import lax
from jax.experimental import pallas as pl
from jax.experimental.pallas import tpu as pltpu
from jax.experimental.pallas import tpu_sc as plsc

N_NODES = 10000
D = 128
EPS = 1e-5

NC = 2
NS = 16
NW = NC * NS
CHUNK = 64
NR = 5
NI = 8
SLAG = 3
N_PAD = 10048
ROW_SPLIT = 632
ROW_LAST = N_PAD - (NS - 1) * ROW_SPLIT


def _make_sc_agg(e_pad: int):
    assert e_pad % (NW * CHUNK) == 0
    cpt = e_pad // (NW * CHUNK)
    ept = cpt * CHUNK
    mesh = plsc.VectorSubcoreMesh(
        core_axis_name="c", subcore_axis_name="s", num_cores=NC, num_subcores=NS
    )

    scratch = (
        [pltpu.VMEM((CHUNK,), jnp.int32) for _ in range(NI)]
        + [pltpu.VMEM((CHUNK,), jnp.int32) for _ in range(NI)]
        + [pltpu.VMEM((CHUNK, D), jnp.float32) for _ in range(NR)]
        + [pltpu.VMEM_SHARED((N_PAD, D), jnp.float32)]
        + [pltpu.SemaphoreType.DMA] * (2 * NI + 2 * NR)
    )

    @functools.partial(
        pl.kernel,
        out_type=jax.ShapeDtypeStruct((NC, N_PAD, D), jnp.float32),
        mesh=mesh,
        scratch_types=scratch,
    )
    def sc_agg(x_hbm, src_hbm, dst_hbm, zeros_hbm, out_hbm, *refs):
        idx_s = refs[0:NI]
        idx_d = refs[NI:2 * NI]
        rows = refs[2 * NI:2 * NI + NR]
        agg_sh = refs[2 * NI + NR]
        sems = refs[2 * NI + NR + 1:]
        isem_s = sems[0:NI]
        isem_d = sems[NI:2 * NI]
        gsem = sems[2 * NI:2 * NI + NR]
        ssem = sems[2 * NI + NR:]

        c = lax.axis_index("c")
        s = lax.axis_index("s")
        wid = c * NS + s
        ebase = wid * ept

        def acc_zero():
            @pl.when(s < NS - 1)
            def _():
                pltpu.sync_copy(zeros_hbm.at[pl.ds(0, ROW_SPLIT)],
                                agg_sh.at[pl.ds(s * ROW_SPLIT, ROW_SPLIT)])

            @pl.when(s == NS - 1)
            def _():
                pltpu.sync_copy(zeros_hbm.at[pl.ds(0, ROW_LAST)],
                                agg_sh.at[pl.ds((NS - 1) * ROW_SPLIT,
                                                ROW_LAST)])

        def acc_out():
            @pl.when(s < NS - 1)
            def _():
                pltpu.sync_copy(agg_sh.at[pl.ds(s * ROW_SPLIT, ROW_SPLIT)],
                                out_hbm.at[c, pl.ds(s * ROW_SPLIT, ROW_SPLIT)])

            @pl.when(s == NS - 1)
            def _():
                pltpu.sync_copy(agg_sh.at[pl.ds((NS - 1) * ROW_SPLIT,
                                                ROW_LAST)],
                                out_hbm.at[c, pl.ds((NS - 1) * ROW_SPLIT,
                                                    ROW_LAST)])

        def i_start(j):
            q = j % NI
            off = ebase + j * CHUNK
            pltpu.async_copy(src_hbm.at[pl.ds(off, CHUNK)], idx_s[q],
                             isem_s[q])
            pltpu.async_copy(dst_hbm.at[pl.ds(off, CHUNK)], idx_d[q],
                             isem_d[q])

        def i_wait(j):
            q = j % NI
            off = ebase + j * CHUNK
            pltpu.make_async_copy(src_hbm.at[pl.ds(off, CHUNK)], idx_s[q],
                                  isem_s[q]).wait()
            pltpu.make_async_copy(dst_hbm.at[pl.ds(off, CHUNK)], idx_d[q],
                                  isem_d[q]).wait()

        def g_start(j):
            pltpu.async_copy(x_hbm.at[idx_s[j % NI]], rows[j % NR],
                             gsem[j % NR])

        def g_wait(j):
            pltpu.make_async_copy(x_hbm.at[idx_s[j % NI]], rows[j % NR],
                                  gsem[j % NR]).wait()

        def s_start(j):
            pltpu.async_copy(rows[j % NR], agg_sh.at[idx_d[j % NI]],
                             ssem[j % NR], add=True)

        def s_wait(j):
            pltpu.make_async_copy(rows[j % NR], agg_sh.at[idx_d[j % NI]],
                                  ssem[j % NR]).wait()

        for j in range(min(SLAG, cpt)):
            i_start(j)
        acc_zero()
        plsc.subcore_barrier()

        for j in range(cpt):
            if j >= NR:
                s_wait(j - NR)
            if j + SLAG < cpt:
                i_start(j + SLAG)
            i_wait(j)
            g_start(j)
            if j >= SLAG:
                g_wait(j - SLAG)
                s_start(j - SLAG)
        for j in range(max(cpt - SLAG, 0), cpt):
            g_wait(j)
            s_start(j)
        for j in range(max(cpt - NR, 0), cpt):
            s_wait(j)

        plsc.subcore_barrier()
        acc_out()

    return sc_agg


def _make_tc_layer(relu: bool):
    def body(p_ref, x_ref, wrel_ref, brel_ref, wroot_ref, gamma_ref, beta_ref,
             o_ref):
        agg = p_ref[0, :N_NODES, :] + p_ref[1, :N_NODES, :]
        h = jnp.dot(agg, wrel_ref[...], preferred_element_type=jnp.float32)
        h = h + jnp.dot(x_ref[...], wroot_ref[...],
                        preferred_element_type=jnp.float32)
        h = h + brel_ref[...]
        mu = jnp.mean(h, axis=0, keepdims=True)
        var = jnp.mean(jnp.square(h - mu), axis=0, keepdims=True)
        hn = (h - mu) * lax.rsqrt(var + EPS) * gamma_ref[...] + beta_ref[...]
        if relu:
            hn = jnp.maximum(hn, 0.0)
        o_ref[...] = hn

    return pl.pallas_call(
        body,
        out_shape=jax.ShapeDtypeStruct((N_NODES, D), jnp.float32),
    )


def kernel(x, edge_index, W_rel1, b_rel1, W_root1, gamma1, beta1,
           W_rel2, b_rel2, W_root2, gamma2, beta2):
    src = edge_index[0].astype(jnp.int32)
    dst = edge_index[1].astype(jnp.int32)
    e = src.shape[0]
    grain = NW * CHUNK
    e_pad = ((e + grain - 1) // grain) * grain
    if e_pad != e:
        npad = e_pad - e
        ar = jnp.arange(npad, dtype=jnp.int32)
        src = jnp.concatenate([src, ar % N_NODES])
        dst = jnp.concatenate([dst, N_NODES + ar % (N_PAD - N_NODES)])

    zeros = jnp.zeros((ROW_SPLIT, D), jnp.float32)

    sc_agg = _make_sc_agg(e_pad)
    tc1 = _make_tc_layer(relu=True)
    tc2 = _make_tc_layer(relu=False)

    b1 = b_rel1.reshape(1, D)
    g1 = gamma1.reshape(1, D)
    be1 = beta1.reshape(1, D)
    b2 = b_rel2.reshape(1, D)
    g2 = gamma2.reshape(1, D)
    be2 = beta2.reshape(1, D)

    p1 = sc_agg(x, src, dst, zeros)
    h1 = tc1(p1, x, W_rel1, b1, W_root1, g1, be1)
    p2 = sc_agg(h1, src, dst, zeros)
    h2 = tc2(p2, h1, W_rel2, b2, W_root2, g2, be2)
    return h2


# speedup vs baseline: 4.2560x; 1.0276x over previous
import functools

import jax
import jax.numpy as jnp
from jax import lax
from jax.experimental import pallas as pl
from jax.experimental.pallas import tpu as pltpu
from jax.experimental.pallas import tpu_sc as plsc

N_NODES = 10000
D = 128
EPS = 1e-5

NC = 2
NS = 16
NW = NC * NS
CHUNK = 64
NR = 5
NI = 9
SLAG = 4
N_PAD = 10048
ROW_SPLIT = 632
ROW_LAST = N_PAD - (NS - 1) * ROW_SPLIT


def _make_sc_agg(e_pad: int):
    assert e_pad % (NW * CHUNK) == 0
    cpt = e_pad // (NW * CHUNK)
    ept = cpt * CHUNK
    mesh = plsc.VectorSubcoreMesh(
        core_axis_name="c", subcore_axis_name="s", num_cores=NC, num_subcores=NS
    )

    scratch = (
        [pltpu.VMEM((CHUNK,), jnp.int32) for _ in range(NI)]
        + [pltpu.VMEM((CHUNK,), jnp.int32) for _ in range(NI)]
        + [pltpu.VMEM((CHUNK, D), jnp.float32) for _ in range(NR)]
        + [pltpu.VMEM_SHARED((N_PAD, D), jnp.float32)]
        + [pltpu.SemaphoreType.DMA] * (2 * NI + 2 * NR)
    )

    @functools.partial(
        pl.kernel,
        out_type=jax.ShapeDtypeStruct((NC, N_PAD, D), jnp.float32),
        mesh=mesh,
        scratch_types=scratch,
    )
    def sc_agg(x_hbm, src_hbm, dst_hbm, zeros_hbm, out_hbm, *refs):
        idx_s = refs[0:NI]
        idx_d = refs[NI:2 * NI]
        rows = refs[2 * NI:2 * NI + NR]
        agg_sh = refs[2 * NI + NR]
        sems = refs[2 * NI + NR + 1:]
        isem_s = sems[0:NI]
        isem_d = sems[NI:2 * NI]
        gsem = sems[2 * NI:2 * NI + NR]
        ssem = sems[2 * NI + NR:]

        c = lax.axis_index("c")
        s = lax.axis_index("s")
        wid = c * NS + s
        ebase = wid * ept

        def acc_zero():
            @pl.when(s < NS - 1)
            def _():
                pltpu.sync_copy(zeros_hbm.at[pl.ds(0, ROW_SPLIT)],
                                agg_sh.at[pl.ds(s * ROW_SPLIT, ROW_SPLIT)])

            @pl.when(s == NS - 1)
            def _():
                pltpu.sync_copy(zeros_hbm.at[pl.ds(0, ROW_LAST)],
                                agg_sh.at[pl.ds((NS - 1) * ROW_SPLIT,
                                                ROW_LAST)])

        def acc_out():
            @pl.when(s < NS - 1)
            def _():
                pltpu.sync_copy(agg_sh.at[pl.ds(s * ROW_SPLIT, ROW_SPLIT)],
                                out_hbm.at[c, pl.ds(s * ROW_SPLIT, ROW_SPLIT)])

            @pl.when(s == NS - 1)
            def _():
                pltpu.sync_copy(agg_sh.at[pl.ds((NS - 1) * ROW_SPLIT,
                                                ROW_LAST)],
                                out_hbm.at[c, pl.ds((NS - 1) * ROW_SPLIT,
                                                    ROW_LAST)])

        def i_start(j):
            q = j % NI
            off = ebase + j * CHUNK
            pltpu.async_copy(src_hbm.at[pl.ds(off, CHUNK)], idx_s[q],
                             isem_s[q])
            pltpu.async_copy(dst_hbm.at[pl.ds(off, CHUNK)], idx_d[q],
                             isem_d[q])

        def i_wait(j):
            q = j % NI
            off = ebase + j * CHUNK
            pltpu.make_async_copy(src_hbm.at[pl.ds(off, CHUNK)], idx_s[q],
                                  isem_s[q]).wait()
            pltpu.make_async_copy(dst_hbm.at[pl.ds(off, CHUNK)], idx_d[q],
                                  isem_d[q]).wait()

        def g_start(j):
            pltpu.async_copy(x_hbm.at[idx_s[j % NI]], rows[j % NR],
                             gsem[j % NR])

        def g_wait(j):
            pltpu.make_async_copy(x_hbm.at[idx_s[j % NI]], rows[j % NR],
                                  gsem[j % NR]).wait()

        def s_start(j):
            pltpu.async_copy(rows[j % NR], agg_sh.at[idx_d[j % NI]],
                             ssem[j % NR], add=True)

        def s_wait(j):
            pltpu.make_async_copy(rows[j % NR], agg_sh.at[idx_d[j % NI]],
                                  ssem[j % NR]).wait()

        for j in range(min(SLAG, cpt)):
            i_start(j)
        acc_zero()
        plsc.subcore_barrier()

        for j in range(cpt):
            if j >= NR:
                s_wait(j - NR)
            if j + SLAG < cpt:
                i_start(j + SLAG)
            i_wait(j)
            g_start(j)
            if j >= SLAG:
                g_wait(j - SLAG)
                s_start(j - SLAG)
        for j in range(max(cpt - SLAG, 0), cpt):
            g_wait(j)
            s_start(j)
        for j in range(max(cpt - NR, 0), cpt):
            s_wait(j)

        plsc.subcore_barrier()
        acc_out()

    return sc_agg


def _make_tc_layer(relu: bool):
    def body(p_ref, x_ref, wrel_ref, brel_ref, wroot_ref, gamma_ref, beta_ref,
             o_ref):
        agg = p_ref[0, :N_NODES, :] + p_ref[1, :N_NODES, :]
        h = jnp.dot(agg, wrel_ref[...], preferred_element_type=jnp.float32)
        h = h + jnp.dot(x_ref[...], wroot_ref[...],
                        preferred_element_type=jnp.float32)
        h = h + brel_ref[...]
        mu = jnp.mean(h, axis=0, keepdims=True)
        var = jnp.mean(jnp.square(h - mu), axis=0, keepdims=True)
        hn = (h - mu) * lax.rsqrt(var + EPS) * gamma_ref[...] + beta_ref[...]
        if relu:
            hn = jnp.maximum(hn, 0.0)
        o_ref[...] = hn

    return pl.pallas_call(
        body,
        out_shape=jax.ShapeDtypeStruct((N_NODES, D), jnp.float32),
    )


def kernel(x, edge_index, W_rel1, b_rel1, W_root1, gamma1, beta1,
           W_rel2, b_rel2, W_root2, gamma2, beta2):
    src = edge_index[0].astype(jnp.int32)
    dst = edge_index[1].astype(jnp.int32)
    e = src.shape[0]
    grain = NW * CHUNK
    e_pad = ((e + grain - 1) // grain) * grain
    if e_pad != e:
        npad = e_pad - e
        ar = jnp.arange(npad, dtype=jnp.int32)
        src = jnp.concatenate([src, ar % N_NODES])
        dst = jnp.concatenate([dst, N_NODES + ar % (N_PAD - N_NODES)])

    zeros = jnp.zeros((ROW_SPLIT, D), jnp.float32)

    sc_agg = _make_sc_agg(e_pad)
    tc1 = _make_tc_layer(relu=True)
    tc2 = _make_tc_layer(relu=False)

    b1 = b_rel1.reshape(1, D)
    g1 = gamma1.reshape(1, D)
    be1 = beta1.reshape(1, D)
    b2 = b_rel2.reshape(1, D)
    g2 = gamma2.reshape(1, D)
    be2 = beta2.reshape(1, D)

    p1 = sc_agg(x, src, dst, zeros)
    h1 = tc1(p1, x, W_rel1, b1, W_root1, g1, be1)
    p2 = sc_agg(h1, src, dst, zeros)
    h2 = tc2(p2, h1, W_rel2, b2, W_root2, g2, be2)
    return h2
